# baseline (speedup 1.0000x reference)
import functools

import jax
import jax.numpy as jnp
from jax.experimental import pallas as pl

_HID = 2048
_NE = 64
_K = 8


def _topk_half(x, wt, b_ref, idx_ref, wgt_ref, base):
    logits = jnp.dot(x, wt, preferred_element_type=jnp.float32)
    logits_t = logits.T
    scores = jax.nn.sigmoid(logits_t)
    biased = scores + b_ref[...]

    row = jax.lax.broadcasted_iota(jnp.int32, biased.shape, 0)
    cur = biased
    picked_i = []
    picked_w = []
    for _ in range(_K):
        m = jnp.max(cur, axis=0, keepdims=True)
        eq = cur == m
        idx = jnp.min(jnp.where(eq, row, _NE), axis=0, keepdims=True)
        onehot = row == idx
        w = jnp.sum(jnp.where(onehot, scores, 0.0), axis=0, keepdims=True)
        picked_i.append(idx)
        picked_w.append(w)
        cur = jnp.where(onehot, -jnp.inf, cur)

    idx_t = jnp.concatenate(picked_i, axis=0)
    wgt_t = jnp.concatenate(picked_w, axis=0)
    denom = jnp.sum(wgt_t, axis=0, keepdims=True) + 1e-20
    bt = x.shape[0]
    idx_ref[pl.ds(base, bt), :] = idx_t.T
    wgt_ref[pl.ds(base, bt), :] = (wgt_t / denom).T


def _router_block(xa_ref, xb_ref, wt_ref, b_ref, idx_ref, wgt_ref):
    wt = wt_ref[...]
    bt = xa_ref.shape[0]
    _topk_half(xa_ref[...], wt, b_ref, idx_ref, wgt_ref, 0)
    _topk_half(xb_ref[...], wt, b_ref, idx_ref, wgt_ref, bt)


@jax.jit
def kernel(hidden_states, weight, e_score_correction_bias):
    x = hidden_states.reshape(-1, _HID).astype(jnp.float32)
    t = x.shape[0]
    bt = 1024
    wt = weight.astype(jnp.float32).T
    bias = e_score_correction_bias.astype(jnp.float32).reshape(_NE, 1)

    grid = (t // (2 * bt),)
    out = pl.pallas_call(
        _router_block,
        grid=grid,
        in_specs=[
            pl.BlockSpec((bt, _HID), lambda i: (2 * i, 0)),
            pl.BlockSpec((bt, _HID), lambda i: (2 * i + 1, 0)),
            pl.BlockSpec((_HID, _NE), lambda i: (0, 0)),
            pl.BlockSpec((_NE, 1), lambda i: (0, 0)),
        ],
        out_specs=[
            pl.BlockSpec((2 * bt, _K), lambda i: (i, 0)),
            pl.BlockSpec((2 * bt, _K), lambda i: (i, 0)),
        ],
        out_shape=[
            jax.ShapeDtypeStruct((t, _K), jnp.int32),
            jax.ShapeDtypeStruct((t, _K), jnp.float32),
        ],
    )(x, x, wt, bias)
    return out[0], out[1]
